# Initial kernel scaffold; baseline (speedup 1.0000x reference)
import jax
import jax.numpy as jnp
from jax.experimental import pallas as pl


def kernel(features, species_idx, emb, W_router, W_experts, b_experts):
    raise NotImplementedError("write your pallas kernel here")


# speedup vs baseline: 6.3291x; 6.3291x over previous
import jax
import jax.numpy as jnp
from jax.experimental import pallas as pl
from jax.experimental.pallas import tpu as pltpu

N_SP_PAD = 128
N_EXP = 16
N_RTD = 8
TILE = 2048


def _body(z_ref, f_ref, emb_ref, wrt_ref, wall_ref, b_ref, o_ref, coef_ref):
    @pl.when(pl.program_id(0) == 0)
    def _():
        embv = emb_ref[...]
        u = embv * jax.nn.sigmoid(embv)
        logits = jnp.dot(u, wrt_ref[...], preferred_element_type=jnp.float32)
        lane = jax.lax.broadcasted_iota(jnp.int32, (N_SP_PAD, N_EXP), 1)
        valid = lane < N_RTD
        lm = jnp.max(jnp.where(valid, logits, jnp.float32(-1e30)),
                     axis=1, keepdims=True)
        ex = jnp.where(valid, jnp.exp(logits - lm), 0.0)
        s = ex / jnp.sum(ex, axis=1, keepdims=True)
        m1 = jnp.max(s, axis=1, keepdims=True)
        i1 = jnp.min(jnp.where((s == m1) & valid, lane, N_EXP),
                     axis=1, keepdims=True)
        msk2 = valid & (lane != i1)
        sm = jnp.where(msk2, s, -1.0)
        m2 = jnp.max(sm, axis=1, keepdims=True)
        i2 = jnp.min(jnp.where(sm == m2, lane, N_EXP), axis=1, keepdims=True)
        keep = (lane == i1) | (lane == i2)
        coef_ref[...] = jnp.where(valid, jnp.where(keep, s, 0.0), 1.0)

    y = jnp.dot(f_ref[...], wall_ref[...],
                preferred_element_type=jnp.float32)
    yb = y + b_ref[0:1, :]
    z = z_ref[...]
    sp = jax.lax.broadcasted_iota(jnp.int32, (z.shape[0], N_SP_PAD), 1)
    onehot = (z == sp).astype(jnp.float32)
    coefg = jnp.dot(onehot, coef_ref[...],
                    preferred_element_type=jnp.float32)
    o_ref[...] = jnp.sum(coefg * yb, axis=1, keepdims=True)


def kernel(features, species_idx, emb, W_router, W_experts, b_experts):
    n, in_f = features.shape
    n_species, embd = emb.shape
    wall = W_experts[:, 0, :].T
    wrt = jnp.zeros((embd, N_EXP), jnp.float32).at[:, :N_RTD].set(W_router.T)
    embp = jnp.zeros((N_SP_PAD, embd), jnp.float32).at[:n_species].set(emb)
    b_rep = jnp.broadcast_to(b_experts.reshape(1, N_EXP), (8, N_EXP))
    z2d = species_idx.astype(jnp.int32).reshape(n, 1)

    out = pl.pallas_call(
        _body,
        grid=(n // TILE,),
        in_specs=[
            pl.BlockSpec((TILE, 1), lambda i: (i, 0)),
            pl.BlockSpec((TILE, in_f), lambda i: (i, 0)),
            pl.BlockSpec((N_SP_PAD, embd), lambda i: (0, 0)),
            pl.BlockSpec((embd, N_EXP), lambda i: (0, 0)),
            pl.BlockSpec((in_f, N_EXP), lambda i: (0, 0)),
            pl.BlockSpec((8, N_EXP), lambda i: (0, 0)),
        ],
        out_specs=pl.BlockSpec((TILE, 1), lambda i: (i, 0)),
        out_shape=jax.ShapeDtypeStruct((n, 1), jnp.float32),
        scratch_shapes=[pltpu.VMEM((N_SP_PAD, N_EXP), jnp.float32)],
    )(z2d, features, embp, wrt, wall, b_rep)
    return out


# speedup vs baseline: 6.4383x; 1.0173x over previous
import jax
import jax.numpy as jnp
from jax.experimental import pallas as pl
from jax.experimental.pallas import tpu as pltpu

N_SP_PAD = 128
N_EXP = 16
N_RTD = 8
TILE = 4096


def _body(z_ref, f_ref, emb_ref, wrt_ref, wall_ref, b_ref, o_ref, coef_ref):
    @pl.when(pl.program_id(0) == 0)
    def _():
        embv = emb_ref[...]
        u = embv * jax.nn.sigmoid(embv)
        logits = jnp.dot(u, wrt_ref[...], preferred_element_type=jnp.float32)
        lane = jax.lax.broadcasted_iota(jnp.int32, (N_SP_PAD, N_EXP), 1)
        valid = lane < N_RTD
        lm = jnp.max(jnp.where(valid, logits, jnp.float32(-1e30)),
                     axis=1, keepdims=True)
        ex = jnp.where(valid, jnp.exp(logits - lm), 0.0)
        s = ex / jnp.sum(ex, axis=1, keepdims=True)
        m1 = jnp.max(s, axis=1, keepdims=True)
        i1 = jnp.min(jnp.where((s == m1) & valid, lane, N_EXP),
                     axis=1, keepdims=True)
        msk2 = valid & (lane != i1)
        sm = jnp.where(msk2, s, -1.0)
        m2 = jnp.max(sm, axis=1, keepdims=True)
        i2 = jnp.min(jnp.where(sm == m2, lane, N_EXP), axis=1, keepdims=True)
        keep = (lane == i1) | (lane == i2)
        coef_ref[...] = jnp.where(valid, jnp.where(keep, s, 0.0), 1.0)

    y = jnp.dot(f_ref[...], wall_ref[...],
                preferred_element_type=jnp.float32)
    yb = y + b_ref[0:1, :]
    z = z_ref[...]
    sp = jax.lax.broadcasted_iota(jnp.int32, (z.shape[0], N_SP_PAD), 1)
    onehot = (z == sp).astype(jnp.float32)
    coefg = jnp.dot(onehot, coef_ref[...],
                    preferred_element_type=jnp.float32)
    o_ref[...] = jnp.sum(coefg * yb, axis=1, keepdims=True)


def kernel(features, species_idx, emb, W_router, W_experts, b_experts):
    n, in_f = features.shape
    n_species, embd = emb.shape
    wall = W_experts[:, 0, :].T
    wrt = jnp.zeros((embd, N_EXP), jnp.float32).at[:, :N_RTD].set(W_router.T)
    embp = jnp.zeros((N_SP_PAD, embd), jnp.float32).at[:n_species].set(emb)
    b_rep = jnp.broadcast_to(b_experts.reshape(1, N_EXP), (8, N_EXP))
    z2d = species_idx.astype(jnp.int32).reshape(n, 1)

    out = pl.pallas_call(
        _body,
        grid=(n // TILE,),
        in_specs=[
            pl.BlockSpec((TILE, 1), lambda i: (i, 0)),
            pl.BlockSpec((TILE, in_f), lambda i: (i, 0)),
            pl.BlockSpec((N_SP_PAD, embd), lambda i: (0, 0)),
            pl.BlockSpec((embd, N_EXP), lambda i: (0, 0)),
            pl.BlockSpec((in_f, N_EXP), lambda i: (0, 0)),
            pl.BlockSpec((8, N_EXP), lambda i: (0, 0)),
        ],
        out_specs=pl.BlockSpec((TILE, 1), lambda i: (i, 0)),
        out_shape=jax.ShapeDtypeStruct((n, 1), jnp.float32),
        scratch_shapes=[pltpu.VMEM((N_SP_PAD, N_EXP), jnp.float32)],
    )(z2d, features, embp, wrt, wall, b_rep)
    return out
